# baseline (speedup 1.0000x reference)
import jax
import jax.numpy as jnp
from jax.experimental import pallas as pl
from jax.experimental.pallas import tpu as pltpu

CONF_THRES = 0.2
IOU_THRES = 0.6
MAX_DET = 100
N0, N1, N2 = 4096, 1024, 256
NTOT = N0 + N1 + N2


def _decode_body(pb0, ob0, cs0, pb1, ob1, cs1, pb2, ob2, cs2, a0, a1, a2,
                 out_ref):
    for (pb, ob, cs, a, n, base) in (
            (pb0, ob0, cs0, a0, N0, 0),
            (pb1, ob1, cs1, a1, N1, N0),
            (pb2, ob2, cs2, a2, N2, N0 + N1)):
        av = a[...]
        ax1 = av[0:1]; ay1 = av[1:2]; ax2 = av[2:3]; ay2 = av[3:4]
        pxy_x = 0.5 * (ax1 + ax2)
        pxy_y = 0.5 * (ay1 + ay2)
        pwh_x = ax2 - ax1
        pwh_y = ay2 - ay1
        pv = pb[0]
        x1 = pxy_x - pv[0:1] * pwh_x
        y1 = pxy_y - pv[1:2] * pwh_y
        x2 = pxy_x + pv[2:3] * pwh_x
        y2 = pxy_y + pv[3:4] * pwh_y
        cx = 0.5 * (x1 + x2); cy = 0.5 * (y1 + y2)
        w = x2 - x1; h = y2 - y1
        hw = 0.5 * w; hh = 0.5 * h
        s = cs[0]
        m = jnp.max(s, axis=0, keepdims=True)
        ri = jax.lax.broadcasted_iota(jnp.int32, (80, n), 0)
        cid = jnp.min(jnp.where(s == m, ri, 128), axis=0, keepdims=True)
        obj = ob[0]
        conf = jnp.sqrt(obj * m)
        sc0 = jnp.where(conf > CONF_THRES, conf, -1.0)
        out_ref[0, 0:1, base:base + n] = cx - hw
        out_ref[0, 1:2, base:base + n] = cy - hh
        out_ref[0, 2:3, base:base + n] = cx + hw
        out_ref[0, 3:4, base:base + n] = cy + hh
        out_ref[0, 4:5, base:base + n] = sc0
        out_ref[0, 5:6, base:base + n] = cid.astype(jnp.float32)
        out_ref[0, 6:8, base:base + n] = jnp.zeros((2, n), jnp.float32)


def _nms_body(dec_ref, out_ref, sc_ref, ox1_ref, oy1_ref, ox2_ref, oy2_ref,
              ar_ref, rx1_ref, ry1_ref, rx2_ref, ry2_ref, cl_ref):
    bs = dec_ref.shape[0]
    x1 = dec_ref[:, 0, :]
    y1 = dec_ref[:, 1, :]
    x2 = dec_ref[:, 2, :]
    y2 = dec_ref[:, 3, :]
    cl = dec_ref[:, 5, :]
    off = cl * 4096.0
    ox1 = x1 + off; oy1 = y1 + off; ox2 = x2 + off; oy2 = y2 + off
    ox1_ref[...] = ox1; oy1_ref[...] = oy1
    ox2_ref[...] = ox2; oy2_ref[...] = oy2
    ar_ref[...] = (ox2 - ox1) * (oy2 - oy1)
    rx1_ref[...] = x1; ry1_ref[...] = y1
    rx2_ref[...] = x2; ry2_ref[...] = y2
    cl_ref[...] = cl
    sc_ref[...] = dec_ref[:, 4, :]

    ii = jax.lax.broadcasted_iota(jnp.int32, (bs, NTOT), 1)
    ti = jax.lax.broadcasted_iota(jnp.int32, (bs, 128), 1)

    def step(t, carry):
        ax1, ay1_, ax2_, ay2_, asc, acl, best = carry
        sc = sc_ref[...]
        isb = sc == best
        idx = jnp.min(jnp.where(isb, ii, jnp.int32(1 << 30)),
                      axis=1, keepdims=True)
        bmask = ii == idx
        valid = best > 0.0

        def sel(ref):
            return jnp.sum(jnp.where(bmask, ref[...], 0.0),
                           axis=1, keepdims=True)

        rx1 = sel(rx1_ref); ry1 = sel(ry1_ref)
        rx2 = sel(rx2_ref); ry2 = sel(ry2_ref)
        bcl = sel(cl_ref)
        boff = bcl * 4096.0
        bx1 = rx1 + boff; by1 = ry1 + boff
        bx2 = rx2 + boff; by2 = ry2 + boff
        ix1 = jnp.maximum(bx1, ox1_ref[...])
        iy1 = jnp.maximum(by1, oy1_ref[...])
        ix2 = jnp.minimum(bx2, ox2_ref[...])
        iy2 = jnp.minimum(by2, oy2_ref[...])
        inter = jnp.maximum(ix2 - ix1, 0.0) * jnp.maximum(iy2 - iy1, 0.0)
        ba = (bx2 - bx1) * (by2 - by1)
        iou = inter / (ba + ar_ref[...] - inter + 1e-9)
        supp = (iou >= IOU_THRES) & valid
        nsc = jnp.where(supp, -1.0, sc)
        nsc = jnp.where(bmask, -1.0, nsc)
        sc_ref[...] = nsc
        nbest = jnp.max(nsc, axis=1, keepdims=True)

        tm = ti == t
        vx1 = jnp.where(valid, rx1, 0.0)
        vy1 = jnp.where(valid, ry1, 0.0)
        vx2 = jnp.where(valid, rx2, 0.0)
        vy2 = jnp.where(valid, ry2, 0.0)
        vsc = jnp.where(valid, best, 0.0)
        vcl = jnp.where(valid, bcl, -1.0)
        return (jnp.where(tm, vx1, ax1), jnp.where(tm, vy1, ay1_),
                jnp.where(tm, vx2, ax2_), jnp.where(tm, vy2, ay2_),
                jnp.where(tm, vsc, asc), jnp.where(tm, vcl, acl), nbest)

    z = jnp.zeros((bs, 128), jnp.float32)
    best0 = jnp.max(sc_ref[...], axis=1, keepdims=True)
    carry = jax.lax.fori_loop(
        0, MAX_DET, step,
        (z, z, z, z, z, jnp.full((bs, 128), -1.0), best0))
    accs = carry[:6]
    for k in range(6):
        out_ref[:, k, :] = accs[k]
    out_ref[:, 6, :] = z
    out_ref[:, 7, :] = z


def kernel(p0_box, p0_ctr, p0_cls, p1_box, p1_ctr, p1_cls, p2_box, p2_ctr,
           p2_cls, a0, a1, a2, image_size):
    bs = p0_box.shape[0]
    pb = [p0_box.reshape(bs, 4, N0), p1_box.reshape(bs, 4, N1),
          p2_box.reshape(bs, 4, N2)]
    ob = [jax.nn.sigmoid(p0_ctr.reshape(bs, 1, N0)),
          jax.nn.sigmoid(p1_ctr.reshape(bs, 1, N1)),
          jax.nn.sigmoid(p2_ctr.reshape(bs, 1, N2))]
    cs = [jax.nn.sigmoid(p0_cls.reshape(bs, 80, N0)),
          jax.nn.sigmoid(p1_cls.reshape(bs, 80, N1)),
          jax.nn.sigmoid(p2_cls.reshape(bs, 80, N2))]
    at = [a0.T, a1.T, a2.T]

    def bspec(c, n):
        return pl.BlockSpec((1, c, n), lambda b: (b, 0, 0))

    def aspec(n):
        return pl.BlockSpec((4, n), lambda b: (0, 0))

    dec = pl.pallas_call(
        _decode_body,
        grid=(bs,),
        in_specs=[bspec(4, N0), bspec(1, N0), bspec(80, N0),
                  bspec(4, N1), bspec(1, N1), bspec(80, N1),
                  bspec(4, N2), bspec(1, N2), bspec(80, N2),
                  aspec(N0), aspec(N1), aspec(N2)],
        out_specs=pl.BlockSpec((1, 8, NTOT), lambda b: (b, 0, 0)),
        out_shape=jax.ShapeDtypeStruct((bs, 8, NTOT), jnp.float32),
    )(pb[0], ob[0], cs[0], pb[1], ob[1], cs[1], pb[2], ob[2], cs[2],
      at[0], at[1], at[2])

    nms = pl.pallas_call(
        _nms_body,
        out_shape=jax.ShapeDtypeStruct((bs, 8, 128), jnp.float32),
        scratch_shapes=[pltpu.VMEM((bs, NTOT), jnp.float32)
                        for _ in range(11)],
    )(dec)

    return jnp.transpose(nms[:, 0:6, 0:MAX_DET], (0, 2, 1))


# speedup vs baseline: 1.0947x; 1.0947x over previous
import jax
import jax.numpy as jnp
from jax.experimental import pallas as pl
from jax.experimental.pallas import tpu as pltpu

CONF_THRES = 0.2
IOU_THRES = 0.6
MAX_DET = 100
N0, N1, N2 = 4096, 1024, 256
NTOT = N0 + N1 + N2


def _decode_body(pb0, ob0, cs0, pb1, ob1, cs1, pb2, ob2, cs2, a0, a1, a2,
                 out_ref):
    for (pb, ob, cs, a, n, base) in (
            (pb0, ob0, cs0, a0, N0, 0),
            (pb1, ob1, cs1, a1, N1, N0),
            (pb2, ob2, cs2, a2, N2, N0 + N1)):
        av = a[...]
        ax1 = av[0:1]; ay1 = av[1:2]; ax2 = av[2:3]; ay2 = av[3:4]
        pxy_x = 0.5 * (ax1 + ax2)
        pxy_y = 0.5 * (ay1 + ay2)
        pwh_x = ax2 - ax1
        pwh_y = ay2 - ay1
        pv = pb[0]
        x1 = pxy_x - pv[0:1] * pwh_x
        y1 = pxy_y - pv[1:2] * pwh_y
        x2 = pxy_x + pv[2:3] * pwh_x
        y2 = pxy_y + pv[3:4] * pwh_y
        cx = 0.5 * (x1 + x2); cy = 0.5 * (y1 + y2)
        w = x2 - x1; h = y2 - y1
        hw = 0.5 * w; hh = 0.5 * h
        s = jax.nn.sigmoid(cs[0])
        m = jnp.max(s, axis=0, keepdims=True)
        ri = jax.lax.broadcasted_iota(jnp.int32, (80, n), 0)
        cid = jnp.min(jnp.where(s == m, ri, 128), axis=0, keepdims=True)
        obj = jax.nn.sigmoid(ob[0])
        conf = jnp.sqrt(obj * m)
        sc0 = jnp.where(conf > CONF_THRES, conf, -1.0)
        out_ref[0, 0:1, base:base + n] = cx - hw
        out_ref[0, 1:2, base:base + n] = cy - hh
        out_ref[0, 2:3, base:base + n] = cx + hw
        out_ref[0, 3:4, base:base + n] = cy + hh
        out_ref[0, 4:5, base:base + n] = sc0
        out_ref[0, 5:6, base:base + n] = cid.astype(jnp.float32)
        out_ref[0, 6:8, base:base + n] = jnp.zeros((2, n), jnp.float32)


def _nms_body(dec_ref, out_ref, sc_ref, ox1_ref, oy1_ref, ox2_ref, oy2_ref,
              ar_ref, rx1_ref, ry1_ref, rx2_ref, ry2_ref, cl_ref):
    bs = dec_ref.shape[0]
    x1 = dec_ref[:, 0, :]
    y1 = dec_ref[:, 1, :]
    x2 = dec_ref[:, 2, :]
    y2 = dec_ref[:, 3, :]
    cl = dec_ref[:, 5, :]
    off = cl * 4096.0
    ox1 = x1 + off; oy1 = y1 + off; ox2 = x2 + off; oy2 = y2 + off
    ox1_ref[...] = ox1; oy1_ref[...] = oy1
    ox2_ref[...] = ox2; oy2_ref[...] = oy2
    ar_ref[...] = (ox2 - ox1) * (oy2 - oy1)
    rx1_ref[...] = x1; ry1_ref[...] = y1
    rx2_ref[...] = x2; ry2_ref[...] = y2
    cl_ref[...] = cl
    sc_ref[...] = dec_ref[:, 4, :]

    ii = jax.lax.broadcasted_iota(jnp.int32, (bs, NTOT), 1)
    ti = jax.lax.broadcasted_iota(jnp.int32, (bs, 128), 1)

    def step(t, carry):
        ax1, ay1_, ax2_, ay2_, asc, acl, best = carry
        sc = sc_ref[...]
        isb = sc == best
        idx = jnp.min(jnp.where(isb, ii, jnp.int32(1 << 30)),
                      axis=1, keepdims=True)
        bmask = ii == idx
        valid = best > 0.0

        def sel(ref):
            return jnp.sum(jnp.where(bmask, ref[...], 0.0),
                           axis=1, keepdims=True)

        rx1 = sel(rx1_ref); ry1 = sel(ry1_ref)
        rx2 = sel(rx2_ref); ry2 = sel(ry2_ref)
        bcl = sel(cl_ref)
        boff = bcl * 4096.0
        bx1 = rx1 + boff; by1 = ry1 + boff
        bx2 = rx2 + boff; by2 = ry2 + boff
        ix1 = jnp.maximum(bx1, ox1_ref[...])
        iy1 = jnp.maximum(by1, oy1_ref[...])
        ix2 = jnp.minimum(bx2, ox2_ref[...])
        iy2 = jnp.minimum(by2, oy2_ref[...])
        inter = jnp.maximum(ix2 - ix1, 0.0) * jnp.maximum(iy2 - iy1, 0.0)
        ba = (bx2 - bx1) * (by2 - by1)
        iou = inter / (ba + ar_ref[...] - inter + 1e-9)
        supp = (iou >= IOU_THRES) & valid
        nsc = jnp.where(supp, -1.0, sc)
        nsc = jnp.where(bmask, -1.0, nsc)
        sc_ref[...] = nsc
        nbest = jnp.max(nsc, axis=1, keepdims=True)

        tm = ti == t
        vx1 = jnp.where(valid, rx1, 0.0)
        vy1 = jnp.where(valid, ry1, 0.0)
        vx2 = jnp.where(valid, rx2, 0.0)
        vy2 = jnp.where(valid, ry2, 0.0)
        vsc = jnp.where(valid, best, 0.0)
        vcl = jnp.where(valid, bcl, -1.0)
        return (jnp.where(tm, vx1, ax1), jnp.where(tm, vy1, ay1_),
                jnp.where(tm, vx2, ax2_), jnp.where(tm, vy2, ay2_),
                jnp.where(tm, vsc, asc), jnp.where(tm, vcl, acl), nbest)

    z = jnp.zeros((bs, 128), jnp.float32)
    best0 = jnp.max(sc_ref[...], axis=1, keepdims=True)
    carry = jax.lax.fori_loop(
        0, MAX_DET, step,
        (z, z, z, z, z, jnp.full((bs, 128), -1.0), best0))
    accs = carry[:6]
    for k in range(6):
        out_ref[:, k, :] = accs[k]
    out_ref[:, 6, :] = z
    out_ref[:, 7, :] = z


def kernel(p0_box, p0_ctr, p0_cls, p1_box, p1_ctr, p1_cls, p2_box, p2_ctr,
           p2_cls, a0, a1, a2, image_size):
    bs = p0_box.shape[0]
    pb = [p0_box.reshape(bs, 4, N0), p1_box.reshape(bs, 4, N1),
          p2_box.reshape(bs, 4, N2)]
    ob = [p0_ctr.reshape(bs, 1, N0), p1_ctr.reshape(bs, 1, N1),
          p2_ctr.reshape(bs, 1, N2)]
    cs = [p0_cls.reshape(bs, 80, N0), p1_cls.reshape(bs, 80, N1),
          p2_cls.reshape(bs, 80, N2)]
    at = [a0.T, a1.T, a2.T]

    def bspec(c, n):
        return pl.BlockSpec((1, c, n), lambda b: (b, 0, 0))

    def aspec(n):
        return pl.BlockSpec((4, n), lambda b: (0, 0))

    dec = pl.pallas_call(
        _decode_body,
        grid=(bs,),
        in_specs=[bspec(4, N0), bspec(1, N0), bspec(80, N0),
                  bspec(4, N1), bspec(1, N1), bspec(80, N1),
                  bspec(4, N2), bspec(1, N2), bspec(80, N2),
                  aspec(N0), aspec(N1), aspec(N2)],
        out_specs=pl.BlockSpec((1, 8, NTOT), lambda b: (b, 0, 0)),
        out_shape=jax.ShapeDtypeStruct((bs, 8, NTOT), jnp.float32),
    )(pb[0], ob[0], cs[0], pb[1], ob[1], cs[1], pb[2], ob[2], cs[2],
      at[0], at[1], at[2])

    nms = pl.pallas_call(
        _nms_body,
        out_shape=jax.ShapeDtypeStruct((bs, 8, 128), jnp.float32),
        scratch_shapes=[pltpu.VMEM((bs, NTOT), jnp.float32)
                        for _ in range(11)],
    )(dec)

    return jnp.transpose(nms[:, 0:6, 0:MAX_DET], (0, 2, 1))


# speedup vs baseline: 1.0965x; 1.0016x over previous
import jax
import jax.numpy as jnp
from jax.experimental import pallas as pl
from jax.experimental.pallas import tpu as pltpu

CONF_THRES = 0.2
IOU_THRES = 0.6
MAX_DET = 100
N0, N1, N2 = 4096, 1024, 256
NTOT = N0 + N1 + N2


def _decode_body(pb0, ob0, cs0, pb1, ob1, cs1, pb2, ob2, cs2, a0, a1, a2,
                 out_ref):
    for (pb, ob, cs, a, n, base) in (
            (pb0, ob0, cs0, a0, N0, 0),
            (pb1, ob1, cs1, a1, N1, N0),
            (pb2, ob2, cs2, a2, N2, N0 + N1)):
        av = a[...]
        ax1 = av[0:1]; ay1 = av[1:2]; ax2 = av[2:3]; ay2 = av[3:4]
        pxy_x = 0.5 * (ax1 + ax2)
        pxy_y = 0.5 * (ay1 + ay2)
        pwh_x = ax2 - ax1
        pwh_y = ay2 - ay1
        pv = pb[0]
        x1 = pxy_x - pv[0:1] * pwh_x
        y1 = pxy_y - pv[1:2] * pwh_y
        x2 = pxy_x + pv[2:3] * pwh_x
        y2 = pxy_y + pv[3:4] * pwh_y
        cx = 0.5 * (x1 + x2); cy = 0.5 * (y1 + y2)
        w = x2 - x1; h = y2 - y1
        hw = 0.5 * w; hh = 0.5 * h
        s = jax.nn.sigmoid(cs[0])
        m = jnp.max(s, axis=0, keepdims=True)
        ri = jax.lax.broadcasted_iota(jnp.int32, (80, n), 0)
        cid = jnp.min(jnp.where(s == m, ri, 128), axis=0, keepdims=True)
        obj = jax.nn.sigmoid(ob[0])
        conf = jnp.sqrt(obj * m)
        sc0 = jnp.where(conf > CONF_THRES, conf, -1.0)
        out_ref[0, 0:1, base:base + n] = cx - hw
        out_ref[0, 1:2, base:base + n] = cy - hh
        out_ref[0, 2:3, base:base + n] = cx + hw
        out_ref[0, 3:4, base:base + n] = cy + hh
        out_ref[0, 4:5, base:base + n] = sc0
        out_ref[0, 5:6, base:base + n] = cid.astype(jnp.float32)
        out_ref[0, 6:8, base:base + n] = jnp.zeros((2, n), jnp.float32)


def _nms_body(dec_ref, out_ref, sc_ref, ox1_ref, oy1_ref, ox2_ref, oy2_ref,
              ar_ref, rx1_ref, ry1_ref, rx2_ref, ry2_ref, cl_ref):
    bs = dec_ref.shape[0]
    x1 = dec_ref[:, 0, :]
    y1 = dec_ref[:, 1, :]
    x2 = dec_ref[:, 2, :]
    y2 = dec_ref[:, 3, :]
    cl = dec_ref[:, 5, :]
    off = cl * 4096.0
    ox1 = x1 + off; oy1 = y1 + off; ox2 = x2 + off; oy2 = y2 + off
    ox1_ref[...] = ox1; oy1_ref[...] = oy1
    ox2_ref[...] = ox2; oy2_ref[...] = oy2
    ar_ref[...] = (ox2 - ox1) * (oy2 - oy1)
    rx1_ref[...] = x1; ry1_ref[...] = y1
    rx2_ref[...] = x2; ry2_ref[...] = y2
    cl_ref[...] = cl
    sc_ref[...] = dec_ref[:, 4, :]

    ii = jax.lax.broadcasted_iota(jnp.int32, (bs, NTOT), 1)
    ti = jax.lax.broadcasted_iota(jnp.int32, (bs, 128), 1)

    def step(t, carry):
        ax1, ay1_, ax2_, ay2_, asc, acl, best = carry
        sc = sc_ref[...]
        isb = sc == best
        idx = jnp.min(jnp.where(isb, ii, jnp.int32(1 << 30)),
                      axis=1, keepdims=True)
        bmask = ii == idx
        valid = best > 0.0

        def sel(ref):
            return jnp.sum(jnp.where(bmask, ref[...], 0.0),
                           axis=1, keepdims=True)

        rx1 = sel(rx1_ref); ry1 = sel(ry1_ref)
        rx2 = sel(rx2_ref); ry2 = sel(ry2_ref)
        bcl = sel(cl_ref)
        boff = bcl * 4096.0
        bx1 = rx1 + boff; by1 = ry1 + boff
        bx2 = rx2 + boff; by2 = ry2 + boff
        ix1 = jnp.maximum(bx1, ox1_ref[...])
        iy1 = jnp.maximum(by1, oy1_ref[...])
        ix2 = jnp.minimum(bx2, ox2_ref[...])
        iy2 = jnp.minimum(by2, oy2_ref[...])
        inter = jnp.maximum(ix2 - ix1, 0.0) * jnp.maximum(iy2 - iy1, 0.0)
        ba = (bx2 - bx1) * (by2 - by1)
        iou = inter / (ba + ar_ref[...] - inter + 1e-9)
        supp = (iou >= IOU_THRES) & valid
        nsc = jnp.where(supp, -1.0, sc)
        nsc = jnp.where(bmask, -1.0, nsc)
        sc_ref[...] = nsc
        nbest = jnp.max(nsc, axis=1, keepdims=True)

        tm = ti == t
        vx1 = jnp.where(valid, rx1, 0.0)
        vy1 = jnp.where(valid, ry1, 0.0)
        vx2 = jnp.where(valid, rx2, 0.0)
        vy2 = jnp.where(valid, ry2, 0.0)
        vsc = jnp.where(valid, best, 0.0)
        vcl = jnp.where(valid, bcl, -1.0)
        return (jnp.where(tm, vx1, ax1), jnp.where(tm, vy1, ay1_),
                jnp.where(tm, vx2, ax2_), jnp.where(tm, vy2, ay2_),
                jnp.where(tm, vsc, asc), jnp.where(tm, vcl, acl), nbest)

    def step2(i, c):
        return step(2 * i + 1, step(2 * i, c))

    z = jnp.zeros((bs, 128), jnp.float32)
    best0 = jnp.max(sc_ref[...], axis=1, keepdims=True)
    carry = jax.lax.fori_loop(
        0, MAX_DET // 2, step2,
        (z, z, z, z, z, jnp.full((bs, 128), -1.0), best0))
    accs = carry[:6]
    for k in range(6):
        out_ref[:, k, :] = accs[k]
    out_ref[:, 6, :] = z
    out_ref[:, 7, :] = z


def kernel(p0_box, p0_ctr, p0_cls, p1_box, p1_ctr, p1_cls, p2_box, p2_ctr,
           p2_cls, a0, a1, a2, image_size):
    bs = p0_box.shape[0]
    pb = [p0_box.reshape(bs, 4, N0), p1_box.reshape(bs, 4, N1),
          p2_box.reshape(bs, 4, N2)]
    ob = [p0_ctr.reshape(bs, 1, N0), p1_ctr.reshape(bs, 1, N1),
          p2_ctr.reshape(bs, 1, N2)]
    cs = [p0_cls.reshape(bs, 80, N0), p1_cls.reshape(bs, 80, N1),
          p2_cls.reshape(bs, 80, N2)]
    at = [a0.T, a1.T, a2.T]

    def bspec(c, n):
        return pl.BlockSpec((1, c, n), lambda b: (b, 0, 0))

    def aspec(n):
        return pl.BlockSpec((4, n), lambda b: (0, 0))

    dec = pl.pallas_call(
        _decode_body,
        grid=(bs,),
        in_specs=[bspec(4, N0), bspec(1, N0), bspec(80, N0),
                  bspec(4, N1), bspec(1, N1), bspec(80, N1),
                  bspec(4, N2), bspec(1, N2), bspec(80, N2),
                  aspec(N0), aspec(N1), aspec(N2)],
        out_specs=pl.BlockSpec((1, 8, NTOT), lambda b: (b, 0, 0)),
        out_shape=jax.ShapeDtypeStruct((bs, 8, NTOT), jnp.float32),
    )(pb[0], ob[0], cs[0], pb[1], ob[1], cs[1], pb[2], ob[2], cs[2],
      at[0], at[1], at[2])

    nms = pl.pallas_call(
        _nms_body,
        out_shape=jax.ShapeDtypeStruct((bs, 8, 128), jnp.float32),
        scratch_shapes=[pltpu.VMEM((bs, NTOT), jnp.float32)
                        for _ in range(11)],
    )(dec)

    return jnp.transpose(nms[:, 0:6, 0:MAX_DET], (0, 2, 1))


# speedup vs baseline: 1.1029x; 1.0058x over previous
import jax
import jax.numpy as jnp
from jax.experimental import pallas as pl
from jax.experimental.pallas import tpu as pltpu

CONF_THRES = 0.2
IOU_THRES = 0.6
MAX_DET = 100
N0, N1, N2 = 4096, 1024, 256
NTOT = N0 + N1 + N2


def _decode_body(pb0, ob0, cs0, pb1, ob1, cs1, pb2, ob2, cs2, a0, a1, a2,
                 out_ref):
    for (pb, ob, cs, a, n, base) in (
            (pb0, ob0, cs0, a0, N0, 0),
            (pb1, ob1, cs1, a1, N1, N0),
            (pb2, ob2, cs2, a2, N2, N0 + N1)):
        av = a[...]
        ax1 = av[0:1]; ay1 = av[1:2]; ax2 = av[2:3]; ay2 = av[3:4]
        pxy_x = 0.5 * (ax1 + ax2)
        pxy_y = 0.5 * (ay1 + ay2)
        pwh_x = ax2 - ax1
        pwh_y = ay2 - ay1
        pv = pb[0]
        x1 = pxy_x - pv[0:1] * pwh_x
        y1 = pxy_y - pv[1:2] * pwh_y
        x2 = pxy_x + pv[2:3] * pwh_x
        y2 = pxy_y + pv[3:4] * pwh_y
        cx = 0.5 * (x1 + x2); cy = 0.5 * (y1 + y2)
        w = x2 - x1; h = y2 - y1
        hw = 0.5 * w; hh = 0.5 * h
        s = jax.nn.sigmoid(cs[0])
        m = jnp.max(s, axis=0, keepdims=True)
        ri = jax.lax.broadcasted_iota(jnp.int32, (80, n), 0)
        cid = jnp.min(jnp.where(s == m, ri, 128), axis=0, keepdims=True)
        obj = jax.nn.sigmoid(ob[0])
        conf = jnp.sqrt(obj * m)
        sc0 = jnp.where(conf > CONF_THRES, conf, -1.0)
        out_ref[0, 0:1, base:base + n] = cx - hw
        out_ref[0, 1:2, base:base + n] = cy - hh
        out_ref[0, 2:3, base:base + n] = cx + hw
        out_ref[0, 3:4, base:base + n] = cy + hh
        out_ref[0, 4:5, base:base + n] = sc0
        out_ref[0, 5:6, base:base + n] = cid.astype(jnp.float32)
        out_ref[0, 6:8, base:base + n] = jnp.zeros((2, n), jnp.float32)


def _nms_body(dec_ref, out_ref, ox1_ref, oy1_ref, ox2_ref, oy2_ref,
              ar_ref, rx1_ref, ry1_ref, rx2_ref, ry2_ref):
    bs = dec_ref.shape[0]
    x1 = dec_ref[:, 0, :]
    y1 = dec_ref[:, 1, :]
    x2 = dec_ref[:, 2, :]
    y2 = dec_ref[:, 3, :]
    cl = dec_ref[:, 5, :]
    off = cl * 4096.0
    ox1 = x1 + off; oy1 = y1 + off; ox2 = x2 + off; oy2 = y2 + off
    ox1_ref[...] = ox1; oy1_ref[...] = oy1
    ox2_ref[...] = ox2; oy2_ref[...] = oy2
    ar_ref[...] = (ox2 - ox1) * (oy2 - oy1)
    rx1_ref[...] = x1; ry1_ref[...] = y1
    rx2_ref[...] = x2; ry2_ref[...] = y2

    ii = jax.lax.broadcasted_iota(jnp.int32, (bs, NTOT), 1)
    key = ii * 128 + cl.astype(jnp.int32)
    ti = jax.lax.broadcasted_iota(jnp.int32, (bs, 128), 1)

    def step(t, carry):
        ax1, ay1_, ax2_, ay2_, asc, acl, best, sc = carry
        isb = sc == best
        kidx = jnp.min(jnp.where(isb, key, jnp.int32(1 << 30)),
                       axis=1, keepdims=True)
        idx = jax.lax.shift_right_logical(kidx, 7)
        bmask = ii == idx
        valid = best > 0.0

        def sel(ref):
            return jnp.sum(jnp.where(bmask, ref[...], 0.0),
                           axis=1, keepdims=True)

        rx1 = sel(rx1_ref); ry1 = sel(ry1_ref)
        rx2 = sel(rx2_ref); ry2 = sel(ry2_ref)
        bcl = (kidx & 127).astype(jnp.float32)
        boff = bcl * 4096.0
        bx1 = rx1 + boff; by1 = ry1 + boff
        bx2 = rx2 + boff; by2 = ry2 + boff
        ix1 = jnp.maximum(bx1, ox1_ref[...])
        iy1 = jnp.maximum(by1, oy1_ref[...])
        ix2 = jnp.minimum(bx2, ox2_ref[...])
        iy2 = jnp.minimum(by2, oy2_ref[...])
        inter = jnp.maximum(ix2 - ix1, 0.0) * jnp.maximum(iy2 - iy1, 0.0)
        ba = (bx2 - bx1) * (by2 - by1)
        iou = inter / (ba + ar_ref[...] - inter + 1e-9)
        supp = (iou >= IOU_THRES) & valid
        nsc = jnp.where(supp, -1.0, sc)
        nsc = jnp.where(bmask, -1.0, nsc)
        nbest = jnp.max(nsc, axis=1, keepdims=True)

        tm = ti == t
        vx1 = jnp.where(valid, rx1, 0.0)
        vy1 = jnp.where(valid, ry1, 0.0)
        vx2 = jnp.where(valid, rx2, 0.0)
        vy2 = jnp.where(valid, ry2, 0.0)
        vsc = jnp.where(valid, best, 0.0)
        vcl = jnp.where(valid, bcl, -1.0)
        return (jnp.where(tm, vx1, ax1), jnp.where(tm, vy1, ay1_),
                jnp.where(tm, vx2, ax2_), jnp.where(tm, vy2, ay2_),
                jnp.where(tm, vsc, asc), jnp.where(tm, vcl, acl), nbest,
                nsc)

    def step2(i, c):
        return step(2 * i + 1, step(2 * i, c))

    z = jnp.zeros((bs, 128), jnp.float32)
    sc0 = dec_ref[:, 4, :]
    best0 = jnp.max(sc0, axis=1, keepdims=True)
    carry = jax.lax.fori_loop(
        0, MAX_DET // 2, step2,
        (z, z, z, z, z, jnp.full((bs, 128), -1.0), best0, sc0))
    accs = carry[:6]
    for k in range(6):
        out_ref[:, k, :] = accs[k]
    out_ref[:, 6, :] = z
    out_ref[:, 7, :] = z


def kernel(p0_box, p0_ctr, p0_cls, p1_box, p1_ctr, p1_cls, p2_box, p2_ctr,
           p2_cls, a0, a1, a2, image_size):
    bs = p0_box.shape[0]
    pb = [p0_box.reshape(bs, 4, N0), p1_box.reshape(bs, 4, N1),
          p2_box.reshape(bs, 4, N2)]
    ob = [p0_ctr.reshape(bs, 1, N0), p1_ctr.reshape(bs, 1, N1),
          p2_ctr.reshape(bs, 1, N2)]
    cs = [p0_cls.reshape(bs, 80, N0), p1_cls.reshape(bs, 80, N1),
          p2_cls.reshape(bs, 80, N2)]
    at = [a0.T, a1.T, a2.T]

    def bspec(c, n):
        return pl.BlockSpec((1, c, n), lambda b: (b, 0, 0))

    def aspec(n):
        return pl.BlockSpec((4, n), lambda b: (0, 0))

    dec = pl.pallas_call(
        _decode_body,
        grid=(bs,),
        in_specs=[bspec(4, N0), bspec(1, N0), bspec(80, N0),
                  bspec(4, N1), bspec(1, N1), bspec(80, N1),
                  bspec(4, N2), bspec(1, N2), bspec(80, N2),
                  aspec(N0), aspec(N1), aspec(N2)],
        out_specs=pl.BlockSpec((1, 8, NTOT), lambda b: (b, 0, 0)),
        out_shape=jax.ShapeDtypeStruct((bs, 8, NTOT), jnp.float32),
    )(pb[0], ob[0], cs[0], pb[1], ob[1], cs[1], pb[2], ob[2], cs[2],
      at[0], at[1], at[2])

    nms = pl.pallas_call(
        _nms_body,
        out_shape=jax.ShapeDtypeStruct((bs, 8, 128), jnp.float32),
        scratch_shapes=[pltpu.VMEM((bs, NTOT), jnp.float32)
                        for _ in range(9)],
    )(dec)

    return jnp.transpose(nms[:, 0:6, 0:MAX_DET], (0, 2, 1))


# speedup vs baseline: 1.1037x; 1.0007x over previous
import jax
import jax.numpy as jnp
from jax.experimental import pallas as pl
from jax.experimental.pallas import tpu as pltpu

CONF_THRES = 0.2
IOU_THRES = 0.6
MAX_DET = 100
N0, N1, N2 = 4096, 1024, 256
NTOT = N0 + N1 + N2


def _decode_body(pb0, ob0, cs0, pb1, ob1, cs1, pb2, ob2, cs2, a0, a1, a2,
                 out_ref):
    for (pb, ob, cs, a, n, base) in (
            (pb0, ob0, cs0, a0, N0, 0),
            (pb1, ob1, cs1, a1, N1, N0),
            (pb2, ob2, cs2, a2, N2, N0 + N1)):
        av = a[...]
        ax1 = av[0:1]; ay1 = av[1:2]; ax2 = av[2:3]; ay2 = av[3:4]
        pxy_x = 0.5 * (ax1 + ax2)
        pxy_y = 0.5 * (ay1 + ay2)
        pwh_x = ax2 - ax1
        pwh_y = ay2 - ay1
        pv = pb[0]
        x1 = pxy_x - pv[0:1] * pwh_x
        y1 = pxy_y - pv[1:2] * pwh_y
        x2 = pxy_x + pv[2:3] * pwh_x
        y2 = pxy_y + pv[3:4] * pwh_y
        cx = 0.5 * (x1 + x2); cy = 0.5 * (y1 + y2)
        w = x2 - x1; h = y2 - y1
        hw = 0.5 * w; hh = 0.5 * h
        s = jax.nn.sigmoid(cs[0])
        m = jnp.max(s, axis=0, keepdims=True)
        ri = jax.lax.broadcasted_iota(jnp.int32, (80, n), 0)
        cid = jnp.min(jnp.where(s == m, ri, 128), axis=0, keepdims=True)
        obj = jax.nn.sigmoid(ob[0])
        conf = jnp.sqrt(obj * m)
        sc0 = jnp.where(conf > CONF_THRES, conf, -1.0)
        out_ref[0, 0:1, base:base + n] = cx - hw
        out_ref[0, 1:2, base:base + n] = cy - hh
        out_ref[0, 2:3, base:base + n] = cx + hw
        out_ref[0, 3:4, base:base + n] = cy + hh
        out_ref[0, 4:5, base:base + n] = sc0
        out_ref[0, 5:6, base:base + n] = cid.astype(jnp.float32)
        out_ref[0, 6:8, base:base + n] = jnp.zeros((2, n), jnp.float32)


def _nms_body(dec_ref, out_ref, ox1_ref, oy1_ref, ox2_ref, oy2_ref,
              ar_ref, rx1_ref, ry1_ref, rx2_ref, ry2_ref, sc_ref):
    bs = dec_ref.shape[0]
    x1 = dec_ref[:, 0, :]
    y1 = dec_ref[:, 1, :]
    x2 = dec_ref[:, 2, :]
    y2 = dec_ref[:, 3, :]
    cl = dec_ref[:, 5, :]
    off = cl * 4096.0
    ox1 = x1 + off; oy1 = y1 + off; ox2 = x2 + off; oy2 = y2 + off
    ox1_ref[...] = ox1; oy1_ref[...] = oy1
    ox2_ref[...] = ox2; oy2_ref[...] = oy2
    ar_ref[...] = (ox2 - ox1) * (oy2 - oy1)
    rx1_ref[...] = x1; ry1_ref[...] = y1
    rx2_ref[...] = x2; ry2_ref[...] = y2

    ii = jax.lax.broadcasted_iota(jnp.int32, (bs, NTOT), 1)
    key = ii * 128 + cl.astype(jnp.int32)
    ti = jax.lax.broadcasted_iota(jnp.int32, (bs, 128), 1)

    def step(t, carry):
        ax1, ay1_, ax2_, ay2_, asc, acl, best = carry
        sc = sc_ref[...]
        isb = sc == best
        kidx = jnp.min(jnp.where(isb, key, jnp.int32(1 << 30)),
                       axis=1, keepdims=True)
        idx = jax.lax.shift_right_logical(kidx, 7)
        bmask = ii == idx
        valid = best > 0.0

        def sel(ref):
            return jnp.sum(jnp.where(bmask, ref[...], 0.0),
                           axis=1, keepdims=True)

        rx1 = sel(rx1_ref); ry1 = sel(ry1_ref)
        rx2 = sel(rx2_ref); ry2 = sel(ry2_ref)
        bcl = (kidx & 127).astype(jnp.float32)
        boff = bcl * 4096.0
        bx1 = rx1 + boff; by1 = ry1 + boff
        bx2 = rx2 + boff; by2 = ry2 + boff
        ix1 = jnp.maximum(bx1, ox1_ref[...])
        iy1 = jnp.maximum(by1, oy1_ref[...])
        ix2 = jnp.minimum(bx2, ox2_ref[...])
        iy2 = jnp.minimum(by2, oy2_ref[...])
        inter = jnp.maximum(ix2 - ix1, 0.0) * jnp.maximum(iy2 - iy1, 0.0)
        ba = (bx2 - bx1) * (by2 - by1)
        iou = inter / (ba + ar_ref[...] - inter + 1e-9)
        supp = (iou >= IOU_THRES) & valid
        nsc = jnp.where(supp, -1.0, sc)
        nsc = jnp.where(bmask, -1.0, nsc)
        sc_ref[...] = nsc
        nbest = jnp.max(nsc, axis=1, keepdims=True)

        tm = ti == t
        vx1 = jnp.where(valid, rx1, 0.0)
        vy1 = jnp.where(valid, ry1, 0.0)
        vx2 = jnp.where(valid, rx2, 0.0)
        vy2 = jnp.where(valid, ry2, 0.0)
        vsc = jnp.where(valid, best, 0.0)
        vcl = jnp.where(valid, bcl, -1.0)
        return (jnp.where(tm, vx1, ax1), jnp.where(tm, vy1, ay1_),
                jnp.where(tm, vx2, ax2_), jnp.where(tm, vy2, ay2_),
                jnp.where(tm, vsc, asc), jnp.where(tm, vcl, acl), nbest)

    def step2(i, c):
        return step(2 * i + 1, step(2 * i, c))

    z = jnp.zeros((bs, 128), jnp.float32)
    sc0 = dec_ref[:, 4, :]
    sc_ref[...] = sc0
    best0 = jnp.max(sc0, axis=1, keepdims=True)
    carry = jax.lax.fori_loop(
        0, MAX_DET // 2, step2,
        (z, z, z, z, z, jnp.full((bs, 128), -1.0), best0))
    accs = carry[:6]
    for k in range(6):
        out_ref[:, k, :] = accs[k]
    out_ref[:, 6, :] = z
    out_ref[:, 7, :] = z


def kernel(p0_box, p0_ctr, p0_cls, p1_box, p1_ctr, p1_cls, p2_box, p2_ctr,
           p2_cls, a0, a1, a2, image_size):
    bs = p0_box.shape[0]
    pb = [p0_box.reshape(bs, 4, N0), p1_box.reshape(bs, 4, N1),
          p2_box.reshape(bs, 4, N2)]
    ob = [p0_ctr.reshape(bs, 1, N0), p1_ctr.reshape(bs, 1, N1),
          p2_ctr.reshape(bs, 1, N2)]
    cs = [p0_cls.reshape(bs, 80, N0), p1_cls.reshape(bs, 80, N1),
          p2_cls.reshape(bs, 80, N2)]
    at = [a0.T, a1.T, a2.T]

    def bspec(c, n):
        return pl.BlockSpec((1, c, n), lambda b: (b, 0, 0))

    def aspec(n):
        return pl.BlockSpec((4, n), lambda b: (0, 0))

    dec = pl.pallas_call(
        _decode_body,
        grid=(bs,),
        in_specs=[bspec(4, N0), bspec(1, N0), bspec(80, N0),
                  bspec(4, N1), bspec(1, N1), bspec(80, N1),
                  bspec(4, N2), bspec(1, N2), bspec(80, N2),
                  aspec(N0), aspec(N1), aspec(N2)],
        out_specs=pl.BlockSpec((1, 8, NTOT), lambda b: (b, 0, 0)),
        out_shape=jax.ShapeDtypeStruct((bs, 8, NTOT), jnp.float32),
    )(pb[0], ob[0], cs[0], pb[1], ob[1], cs[1], pb[2], ob[2], cs[2],
      at[0], at[1], at[2])

    nms = pl.pallas_call(
        _nms_body,
        out_shape=jax.ShapeDtypeStruct((bs, 8, 128), jnp.float32),
        scratch_shapes=[pltpu.VMEM((bs, NTOT), jnp.float32)
                        for _ in range(10)],
    )(dec)

    return jnp.transpose(nms[:, 0:6, 0:MAX_DET], (0, 2, 1))


# speedup vs baseline: 1.1233x; 1.0178x over previous
import jax
import jax.numpy as jnp
from jax.experimental import pallas as pl
from jax.experimental.pallas import tpu as pltpu

CONF_THRES = 0.2
IOU_THRES = 0.6
MAX_DET = 100
N0, N1, N2 = 4096, 1024, 256
NTOT = N0 + N1 + N2


def _decode_body(pb0, ob0, cs0, pb1, ob1, cs1, pb2, ob2, cs2, a0, a1, a2,
                 out_ref):
    for (pb, ob, cs, a, n, base) in (
            (pb0, ob0, cs0, a0, N0, 0),
            (pb1, ob1, cs1, a1, N1, N0),
            (pb2, ob2, cs2, a2, N2, N0 + N1)):
        av = a[...]
        ax1 = av[0:1]; ay1 = av[1:2]; ax2 = av[2:3]; ay2 = av[3:4]
        pxy_x = 0.5 * (ax1 + ax2)
        pxy_y = 0.5 * (ay1 + ay2)
        pwh_x = ax2 - ax1
        pwh_y = ay2 - ay1
        pv = pb[0]
        x1 = pxy_x - pv[0:1] * pwh_x
        y1 = pxy_y - pv[1:2] * pwh_y
        x2 = pxy_x + pv[2:3] * pwh_x
        y2 = pxy_y + pv[3:4] * pwh_y
        cx = 0.5 * (x1 + x2); cy = 0.5 * (y1 + y2)
        w = x2 - x1; h = y2 - y1
        hw = 0.5 * w; hh = 0.5 * h
        s = jax.nn.sigmoid(cs[0])
        m = jnp.max(s, axis=0, keepdims=True)
        ri = jax.lax.broadcasted_iota(jnp.int32, (80, n), 0)
        cid = jnp.min(jnp.where(s == m, ri, 128), axis=0, keepdims=True)
        obj = jax.nn.sigmoid(ob[0])
        conf = jnp.sqrt(obj * m)
        sc0 = jnp.where(conf > CONF_THRES, conf, -1.0)
        out_ref[0, 0:1, base:base + n] = cx - hw
        out_ref[0, 1:2, base:base + n] = cy - hh
        out_ref[0, 2:3, base:base + n] = cx + hw
        out_ref[0, 3:4, base:base + n] = cy + hh
        out_ref[0, 4:5, base:base + n] = sc0
        out_ref[0, 5:6, base:base + n] = cid.astype(jnp.float32)
        out_ref[0, 6:8, base:base + n] = jnp.zeros((2, n), jnp.float32)


def _nms_body(dec_ref, out_ref, ox1_ref, oy1_ref, ox2_ref, oy2_ref,
              ar_ref, rx1_ref, ry1_ref, rx2_ref, ry2_ref, sc_ref):
    bs = dec_ref.shape[0]
    x1 = dec_ref[:, 0, :]
    y1 = dec_ref[:, 1, :]
    x2 = dec_ref[:, 2, :]
    y2 = dec_ref[:, 3, :]
    cl = dec_ref[:, 5, :]
    off = cl * 4096.0
    ox1 = x1 + off; oy1 = y1 + off; ox2 = x2 + off; oy2 = y2 + off
    ox1_ref[...] = ox1; oy1_ref[...] = oy1
    ox2_ref[...] = ox2; oy2_ref[...] = oy2
    ar_ref[...] = (ox2 - ox1) * (oy2 - oy1)
    rx1_ref[...] = x1; ry1_ref[...] = y1
    rx2_ref[...] = x2; ry2_ref[...] = y2

    ii = jax.lax.broadcasted_iota(jnp.int32, (bs, NTOT), 1)
    key = ii * 128 + cl.astype(jnp.int32)
    ti = jax.lax.broadcasted_iota(jnp.int32, (bs, 128), 1)

    def _tree(v, op):
        a = op(v[:, 0:2688], v[:, 2688:5376])
        t1 = a[:, 2560:2688]
        b = op(a[:, 0:1280], a[:, 1280:2560])
        c = op(b[:, 0:640], b[:, 640:1280])
        t2 = c[:, 512:640]
        d = op(c[:, 0:256], c[:, 256:512])
        e = op(d[:, 0:128], d[:, 128:256])
        return op(op(e, t1), t2)

    def step(t, carry):
        ax1, ay1_, ax2_, ay2_, asc, acl, best = carry
        sc = sc_ref[...]
        isb = sc == best
        kidx = jnp.min(_tree(jnp.where(isb, key, jnp.int32(1 << 30)),
                             jnp.minimum),
                       axis=1, keepdims=True)
        idx = jax.lax.shift_right_logical(kidx, 7)
        bmask = ii == idx
        valid = best > 0.0

        def sel(ref):
            return jnp.sum(_tree(jnp.where(bmask, ref[...], 0.0), jnp.add),
                           axis=1, keepdims=True)

        rx1 = sel(rx1_ref); ry1 = sel(ry1_ref)
        rx2 = sel(rx2_ref); ry2 = sel(ry2_ref)
        bcl = (kidx & 127).astype(jnp.float32)
        boff = bcl * 4096.0
        bx1 = rx1 + boff; by1 = ry1 + boff
        bx2 = rx2 + boff; by2 = ry2 + boff
        ix1 = jnp.maximum(bx1, ox1_ref[...])
        iy1 = jnp.maximum(by1, oy1_ref[...])
        ix2 = jnp.minimum(bx2, ox2_ref[...])
        iy2 = jnp.minimum(by2, oy2_ref[...])
        inter = jnp.maximum(ix2 - ix1, 0.0) * jnp.maximum(iy2 - iy1, 0.0)
        ba = (bx2 - bx1) * (by2 - by1)
        iou = inter / (ba + ar_ref[...] - inter + 1e-9)
        supp = (iou >= IOU_THRES) & valid
        nsc = jnp.where(supp, -1.0, sc)
        nsc = jnp.where(bmask, -1.0, nsc)
        sc_ref[...] = nsc
        nbest = jnp.max(_tree(nsc, jnp.maximum), axis=1, keepdims=True)

        tm = ti == t
        vx1 = jnp.where(valid, rx1, 0.0)
        vy1 = jnp.where(valid, ry1, 0.0)
        vx2 = jnp.where(valid, rx2, 0.0)
        vy2 = jnp.where(valid, ry2, 0.0)
        vsc = jnp.where(valid, best, 0.0)
        vcl = jnp.where(valid, bcl, -1.0)
        return (jnp.where(tm, vx1, ax1), jnp.where(tm, vy1, ay1_),
                jnp.where(tm, vx2, ax2_), jnp.where(tm, vy2, ay2_),
                jnp.where(tm, vsc, asc), jnp.where(tm, vcl, acl), nbest)

    def step2(i, c):
        return step(2 * i + 1, step(2 * i, c))

    z = jnp.zeros((bs, 128), jnp.float32)
    sc0 = dec_ref[:, 4, :]
    sc_ref[...] = sc0
    best0 = jnp.max(sc0, axis=1, keepdims=True)
    carry = jax.lax.fori_loop(
        0, MAX_DET // 2, step2,
        (z, z, z, z, z, jnp.full((bs, 128), -1.0), best0))
    accs = carry[:6]
    for k in range(6):
        out_ref[:, k, :] = accs[k]
    out_ref[:, 6, :] = z
    out_ref[:, 7, :] = z


def kernel(p0_box, p0_ctr, p0_cls, p1_box, p1_ctr, p1_cls, p2_box, p2_ctr,
           p2_cls, a0, a1, a2, image_size):
    bs = p0_box.shape[0]
    pb = [p0_box.reshape(bs, 4, N0), p1_box.reshape(bs, 4, N1),
          p2_box.reshape(bs, 4, N2)]
    ob = [p0_ctr.reshape(bs, 1, N0), p1_ctr.reshape(bs, 1, N1),
          p2_ctr.reshape(bs, 1, N2)]
    cs = [p0_cls.reshape(bs, 80, N0), p1_cls.reshape(bs, 80, N1),
          p2_cls.reshape(bs, 80, N2)]
    at = [a0.T, a1.T, a2.T]

    def bspec(c, n):
        return pl.BlockSpec((1, c, n), lambda b: (b, 0, 0))

    def aspec(n):
        return pl.BlockSpec((4, n), lambda b: (0, 0))

    dec = pl.pallas_call(
        _decode_body,
        grid=(bs,),
        in_specs=[bspec(4, N0), bspec(1, N0), bspec(80, N0),
                  bspec(4, N1), bspec(1, N1), bspec(80, N1),
                  bspec(4, N2), bspec(1, N2), bspec(80, N2),
                  aspec(N0), aspec(N1), aspec(N2)],
        out_specs=pl.BlockSpec((1, 8, NTOT), lambda b: (b, 0, 0)),
        out_shape=jax.ShapeDtypeStruct((bs, 8, NTOT), jnp.float32),
    )(pb[0], ob[0], cs[0], pb[1], ob[1], cs[1], pb[2], ob[2], cs[2],
      at[0], at[1], at[2])

    nms = pl.pallas_call(
        _nms_body,
        out_shape=jax.ShapeDtypeStruct((bs, 8, 128), jnp.float32),
        scratch_shapes=[pltpu.VMEM((bs, NTOT), jnp.float32)
                        for _ in range(10)],
    )(dec)

    return jnp.transpose(nms[:, 0:6, 0:MAX_DET], (0, 2, 1))
